# Initial kernel scaffold; baseline (speedup 1.0000x reference)
import functools

import jax
import jax.numpy as jnp
from jax import lax
from jax.experimental import pallas as pl
from jax.experimental.pallas import tpu as pltpu
from jax.experimental.pallas import tpu_sc as plsc

N_NODES = 10000
NNZ = 320000
D = 128

NC = 2
NS = 16
NW = NC * NS

PER_TILE = NNZ // NW
BATCH = 80
NUM_BATCHES = PER_TILE // BATCH
ROWS_PER_TILE = N_NODES // NS


def _sc_partial_pool(row_idx, values):
    mesh = plsc.VectorSubcoreMesh(core_axis_name="c", subcore_axis_name="s")

    @functools.partial(
        pl.kernel,
        mesh=mesh,
        out_type=jax.ShapeDtypeStruct((NC, N_NODES, D), jnp.float32),
        scratch_types=[
            pltpu.VMEM((BATCH,), jnp.int32),
            pltpu.VMEM((BATCH, D), jnp.float32),
            pltpu.VMEM((ROWS_PER_TILE, D), jnp.float32),
            pltpu.VMEM_SHARED((N_NODES, D), jnp.float32),
        ],
    )
    def k(idx_hbm, vals_hbm, out_hbm, idx_v, rows_v, zero_v, acc_s):
        c = lax.axis_index("c")
        s = lax.axis_index("s")
        wid = c * NS + s

        z16 = jnp.zeros((16,), jnp.float32)

        def zrow(i, carry):
            for j in range(D // 16):
                zero_v[i, pl.ds(j * 16, 16)] = z16
            return carry

        lax.fori_loop(0, ROWS_PER_TILE, zrow, 0)
        pltpu.sync_copy(zero_v, acc_s.at[pl.ds(s * ROWS_PER_TILE, ROWS_PER_TILE)])
        plsc.subcore_barrier()

        base = wid * PER_TILE

        def step(b, carry):
            off = base + b * BATCH
            pltpu.sync_copy(idx_hbm.at[pl.ds(off, BATCH)], idx_v)
            pltpu.sync_copy(vals_hbm.at[pl.ds(off, BATCH)], rows_v)
            pltpu.sync_copy(rows_v, acc_s.at[idx_v], add=True)
            return carry

        lax.fori_loop(0, NUM_BATCHES, step, 0)
        plsc.subcore_barrier()

        r0 = s * ROWS_PER_TILE
        pltpu.sync_copy(
            acc_s.at[pl.ds(r0, ROWS_PER_TILE)],
            out_hbm.at[c, pl.ds(r0, ROWS_PER_TILE)],
        )

    return k(row_idx, values)


def _combine_kernel(p_ref, o_ref):
    o_ref[...] = p_ref[0] + p_ref[1]


def kernel(indices, values):
    row_idx = indices[0].astype(jnp.int32)
    partial = _sc_partial_pool(row_idx, values)
    blk = 1000
    return pl.pallas_call(
        _combine_kernel,
        out_shape=jax.ShapeDtypeStruct((N_NODES, D), jnp.float32),
        grid=(N_NODES // blk,),
        in_specs=[pl.BlockSpec((NC, blk, D), lambda i: (0, i, 0))],
        out_specs=pl.BlockSpec((blk, D), lambda i: (i, 0)),
    )(partial)


# speedup vs baseline: 4.5669x; 4.5669x over previous
import functools

import jax
import jax.numpy as jnp
from jax import lax
from jax.experimental import pallas as pl
from jax.experimental.pallas import tpu as pltpu
from jax.experimental.pallas import tpu_sc as plsc

N_NODES = 10000
NNZ = 320000
D = 128

NC = 2
NS = 16

HALF = N_NODES // NC
ACC_ROWS = HALF + NS

PER_TILE = NNZ // NS
BATCH = 80
NUM_BATCHES = PER_TILE // BATCH
NBUF = 2

ROW_CHUNK = 312
ZERO_TAIL = ACC_ROWS - NS * ROW_CHUNK
OUT_TAIL = HALF - NS * ROW_CHUNK


def _sc_pool(row_idx, values):
    mesh = plsc.VectorSubcoreMesh(core_axis_name="c", subcore_axis_name="s")

    @functools.partial(
        pl.kernel,
        mesh=mesh,
        out_type=jax.ShapeDtypeStruct((N_NODES, D), jnp.float32),
        scratch_types=[
            pltpu.VMEM((NBUF, BATCH), jnp.int32),
            pltpu.VMEM((BATCH,), jnp.int32),
            pltpu.VMEM((NBUF, BATCH, D), jnp.float32),
            pltpu.VMEM((ROW_CHUNK, D), jnp.float32),
            pltpu.VMEM_SHARED((ACC_ROWS, D), jnp.float32),
            pltpu.SemaphoreType.DMA,
            pltpu.SemaphoreType.DMA,
            pltpu.SemaphoreType.DMA,
            pltpu.SemaphoreType.DMA,
        ],
    )
    def k(idx_hbm, vals_hbm, out_hbm, idx_v, dst_v, rows_v, zero_v, acc_s,
          sem_i0, sem_i1, sem_v0, sem_v1):
        sem_i = (sem_i0, sem_i1)
        sem_v = (sem_v0, sem_v1)
        c = lax.axis_index("c")
        s = lax.axis_index("s")
        lo = c * HALF
        trash = HALF + s

        z16 = jnp.zeros((16,), jnp.float32)

        def zrow(i, carry):
            for j in range(D // 16):
                zero_v[i, pl.ds(j * 16, 16)] = z16
            return carry

        lax.fori_loop(0, ROW_CHUNK, zrow, 0)
        pltpu.sync_copy(zero_v, acc_s.at[pl.ds(s * ROW_CHUNK, ROW_CHUNK)])

        @pl.when(s == NS - 1)
        def _zero_tail():
            pltpu.sync_copy(
                zero_v.at[pl.ds(0, ZERO_TAIL)],
                acc_s.at[pl.ds(NS * ROW_CHUNK, ZERO_TAIL)],
            )

        plsc.subcore_barrier()

        base = s * PER_TILE

        for t in range(NBUF):
            off = base + t * BATCH
            pltpu.async_copy(idx_hbm.at[pl.ds(off, BATCH)], idx_v.at[t], sem_i[t])
            pltpu.async_copy(vals_hbm.at[pl.ds(off, BATCH)], rows_v.at[t], sem_v[t])

        def body(g, carry):
            for t in range(NBUF):
                b = g * NBUF + t
                pltpu.make_async_copy(
                    idx_hbm.at[pl.ds(0, BATCH)], idx_v.at[t], sem_i[t]
                ).wait()
                pltpu.make_async_copy(
                    vals_hbm.at[pl.ds(0, BATCH)], rows_v.at[t], sem_v[t]
                ).wait()

                for q in range(BATCH // 16):
                    v = idx_v[t, pl.ds(q * 16, 16)]
                    inh = (v >= lo) & (v < lo + HALF)
                    dst_v[pl.ds(q * 16, 16)] = jnp.where(inh, v - lo, trash)

                pltpu.sync_copy(rows_v.at[t], acc_s.at[dst_v], add=True)

                nb = b + NBUF

                @pl.when(nb < NUM_BATCHES)
                def _start_next():
                    off2 = base + nb * BATCH
                    pltpu.async_copy(
                        idx_hbm.at[pl.ds(off2, BATCH)], idx_v.at[t], sem_i[t]
                    )
                    pltpu.async_copy(
                        vals_hbm.at[pl.ds(off2, BATCH)], rows_v.at[t], sem_v[t]
                    )

            return carry

        lax.fori_loop(0, NUM_BATCHES // NBUF, body, 0)
        plsc.subcore_barrier()

        r0 = s * ROW_CHUNK
        pltpu.sync_copy(
            acc_s.at[pl.ds(r0, ROW_CHUNK)],
            out_hbm.at[pl.ds(lo + r0, ROW_CHUNK)],
        )

        @pl.when(s == NS - 1)
        def _write_tail():
            pltpu.sync_copy(
                acc_s.at[pl.ds(NS * ROW_CHUNK, OUT_TAIL)],
                out_hbm.at[pl.ds(lo + NS * ROW_CHUNK, OUT_TAIL)],
            )

    return k(row_idx, values)


def kernel(indices, values):
    row_idx = indices[0].astype(jnp.int32)
    return _sc_pool(row_idx, values)


# speedup vs baseline: 5.6526x; 1.2377x over previous
import functools

import jax
import jax.numpy as jnp
from jax import lax
from jax.experimental import pallas as pl
from jax.experimental.pallas import tpu as pltpu
from jax.experimental.pallas import tpu_sc as plsc

N_NODES = 10000
NNZ = 320000
D = 128

NC = 2
NS = 16

HALF = N_NODES // NC
ACC_ROWS = HALF + NS

PER_TILE = NNZ // NS
BATCH = 80
NUM_BATCHES = PER_TILE // BATCH
NBUF = 3

ROW_CHUNK = 312
ZERO_TAIL = ACC_ROWS - NS * ROW_CHUNK
OUT_TAIL = HALF - NS * ROW_CHUNK


def _sc_pool(row_idx, values):
    mesh = plsc.VectorSubcoreMesh(core_axis_name="c", subcore_axis_name="s")

    @functools.partial(
        pl.kernel,
        mesh=mesh,
        out_type=jax.ShapeDtypeStruct((N_NODES, D), jnp.float32),
        scratch_types=[
            pltpu.VMEM((PER_TILE,), jnp.int32),
            pltpu.VMEM((BATCH,), jnp.int32),
            pltpu.VMEM((NBUF, BATCH, D), jnp.float32),
            pltpu.VMEM((ROW_CHUNK, D), jnp.float32),
            pltpu.VMEM_SHARED((ACC_ROWS, D), jnp.float32),
            pltpu.SemaphoreType.DMA,
            pltpu.SemaphoreType.DMA,
            pltpu.SemaphoreType.DMA,
            pltpu.SemaphoreType.DMA,
        ],
    )
    def k(idx_hbm, vals_hbm, out_hbm, idx_all, dst_v, rows_v, zero_v, acc_s,
          sem_i, sem_v0, sem_v1, sem_v2):
        sem_v = (sem_v0, sem_v1, sem_v2)
        c = lax.axis_index("c")
        s = lax.axis_index("s")
        lo = c * HALF
        trash = HALF + s

        base = s * PER_TILE

        idx_cd = pltpu.async_copy(
            idx_hbm.at[pl.ds(base, PER_TILE)], idx_all, sem_i
        )

        z16 = jnp.zeros((16,), jnp.float32)

        def zrow(i, carry):
            for j in range(D // 16):
                zero_v[i, pl.ds(j * 16, 16)] = z16
            return carry

        lax.fori_loop(0, ROW_CHUNK, zrow, 0)
        pltpu.sync_copy(zero_v, acc_s.at[pl.ds(s * ROW_CHUNK, ROW_CHUNK)])

        @pl.when(s == NS - 1)
        def _zero_tail():
            pltpu.sync_copy(
                zero_v.at[pl.ds(0, ZERO_TAIL)],
                acc_s.at[pl.ds(NS * ROW_CHUNK, ZERO_TAIL)],
            )

        plsc.subcore_barrier()
        idx_cd.wait()

        for t in range(NBUF):
            off = base + t * BATCH
            pltpu.async_copy(vals_hbm.at[pl.ds(off, BATCH)], rows_v.at[t], sem_v[t])

        def process(b, t, issue_next):
            for q in range(BATCH // 16):
                v = idx_all[pl.ds(b * BATCH + q * 16, 16)]
                inh = (v >= lo) & (v < lo + HALF)
                dst_v[pl.ds(q * 16, 16)] = jnp.where(inh, v - lo, trash)

            pltpu.make_async_copy(
                vals_hbm.at[pl.ds(0, BATCH)], rows_v.at[t], sem_v[t]
            ).wait()

            pltpu.sync_copy(rows_v.at[t], acc_s.at[dst_v], add=True)

            if issue_next:
                nb = b + NBUF

                @pl.when(nb < NUM_BATCHES)
                def _start_next():
                    off2 = base + nb * BATCH
                    pltpu.async_copy(
                        vals_hbm.at[pl.ds(off2, BATCH)], rows_v.at[t], sem_v[t]
                    )

        def body(g, carry):
            for t in range(NBUF):
                process(g * NBUF + t, t, True)
            return carry

        full_groups = NUM_BATCHES // NBUF
        lax.fori_loop(0, full_groups, body, 0)
        for b in range(full_groups * NBUF, NUM_BATCHES):
            process(b, b % NBUF, False)
        plsc.subcore_barrier()

        r0 = s * ROW_CHUNK
        pltpu.sync_copy(
            acc_s.at[pl.ds(r0, ROW_CHUNK)],
            out_hbm.at[pl.ds(lo + r0, ROW_CHUNK)],
        )

        @pl.when(s == NS - 1)
        def _write_tail():
            pltpu.sync_copy(
                acc_s.at[pl.ds(NS * ROW_CHUNK, OUT_TAIL)],
                out_hbm.at[pl.ds(lo + NS * ROW_CHUNK, OUT_TAIL)],
            )

    return k(row_idx, values)


def kernel(indices, values):
    row_idx = indices[0].astype(jnp.int32)
    return _sc_pool(row_idx, values)
